# baseline (speedup 1.0000x reference)
import jax
import jax.numpy as jnp
from jax.experimental import pallas as pl

_ACQ = 1
_B, _S, _D = 1024, 4, 2048
_BR = 128


def _copy_body(in_ref, out_ref):
    out_ref[...] = in_ref[...]


def kernel(inputs):
    flat = inputs.reshape(_B, _S * _D)
    return pl.pallas_call(
        _copy_body,
        grid=(_B // _BR,),
        in_specs=[pl.BlockSpec((_BR, _D), lambda i: (i, _ACQ))],
        out_specs=pl.BlockSpec((_BR, _D), lambda i: (i, 0)),
        out_shape=jax.ShapeDtypeStruct((_B, _D), jnp.float32),
    )(flat)


# speedup vs baseline: 1.4768x; 1.4768x over previous
import functools

import jax
import jax.numpy as jnp
from jax import lax
from jax.experimental import pallas as pl
from jax.experimental.pallas import tpu as pltpu
from jax.experimental.pallas import tpu_sc as plsc

_ACQ = 1
_B, _S, _D = 1024, 4, 2048
_NC, _NS = 2, 16
_NW = _NC * _NS
_RPW = _B // _NW
_NCH = 4
_CH = _RPW // _NCH


@functools.partial(
    pl.kernel,
    mesh=plsc.VectorSubcoreMesh(core_axis_name="c", subcore_axis_name="s"),
    out_type=jax.ShapeDtypeStruct((_B, 1, _D), jnp.float32),
    scratch_types=(
        [pltpu.VMEM((_CH, 1, _D), jnp.float32) for _ in range(_NCH)]
        + [pltpu.SemaphoreType.DMA for _ in range(2 * _NCH)]
    ),
)
def _split(in_hbm, out_hbm, *scratch):
    bufs = scratch[:_NCH]
    gsems = scratch[_NCH : 2 * _NCH]
    ssems = scratch[2 * _NCH :]
    wid = lax.axis_index("s") * _NC + lax.axis_index("c")
    base = wid * _RPW

    def gcopy(k):
        return pltpu.make_async_copy(
            in_hbm.at[pl.ds(base + k * _CH, _CH), pl.ds(_ACQ, 1), :],
            bufs[k],
            gsems[k],
        )

    def scopy(k):
        return pltpu.make_async_copy(
            bufs[k],
            out_hbm.at[pl.ds(base + k * _CH, _CH)],
            ssems[k],
        )

    for k in range(_NCH):
        gcopy(k).start()
    for k in range(_NCH):
        gcopy(k).wait()
        scopy(k).start()
    for k in range(_NCH):
        scopy(k).wait()


def kernel(inputs):
    return _split(inputs).reshape(_B, _D)


# speedup vs baseline: 1.4777x; 1.0006x over previous
import functools

import jax
import jax.numpy as jnp
from jax import lax
from jax.experimental import pallas as pl
from jax.experimental.pallas import tpu as pltpu
from jax.experimental.pallas import tpu_sc as plsc

_ACQ = 1
_B, _S, _D = 1024, 4, 2048
_NC, _NS = 2, 16
_NW = _NC * _NS
_RPW = _B // _NW
_NCH = 2
_CH = _RPW // _NCH


@functools.partial(
    pl.kernel,
    mesh=plsc.VectorSubcoreMesh(core_axis_name="c", subcore_axis_name="s"),
    out_type=jax.ShapeDtypeStruct((_B, 1, _D), jnp.float32),
    scratch_types=(
        [pltpu.VMEM((_CH, 1, _D), jnp.float32) for _ in range(_NCH)]
        + [pltpu.SemaphoreType.DMA for _ in range(2 * _NCH)]
    ),
)
def _split(in_hbm, out_hbm, *scratch):
    bufs = scratch[:_NCH]
    gsems = scratch[_NCH : 2 * _NCH]
    ssems = scratch[2 * _NCH :]
    wid = lax.axis_index("s") * _NC + lax.axis_index("c")
    base = wid * _RPW

    def gcopy(k):
        return pltpu.make_async_copy(
            in_hbm.at[pl.ds(base + k * _CH, _CH), pl.ds(_ACQ, 1), :],
            bufs[k],
            gsems[k],
        )

    def scopy(k):
        return pltpu.make_async_copy(
            bufs[k],
            out_hbm.at[pl.ds(base + k * _CH, _CH)],
            ssems[k],
        )

    for k in range(_NCH):
        gcopy(k).start()
    for k in range(_NCH):
        gcopy(k).wait()
        scopy(k).start()
    for k in range(_NCH):
        scopy(k).wait()


def kernel(inputs):
    return _split(inputs).reshape(_B, _D)


# speedup vs baseline: 1.4860x; 1.0056x over previous
import functools

import jax
import jax.numpy as jnp
from jax import lax
from jax.experimental import pallas as pl
from jax.experimental.pallas import tpu as pltpu
from jax.experimental.pallas import tpu_sc as plsc

_ACQ = 1
_B, _S, _D = 1024, 4, 2048
_NC, _NS = 2, 16
_NW = _NC * _NS
_RPW = _B // _NW


@functools.partial(
    pl.kernel,
    mesh=plsc.VectorSubcoreMesh(core_axis_name="c", subcore_axis_name="s"),
    out_type=jax.ShapeDtypeStruct((_B, 1, _D), jnp.float32),
    scratch_types=[pltpu.VMEM((_RPW, 1, _D), jnp.float32)],
)
def _split(in_hbm, out_hbm, buf_v):
    wid = lax.axis_index("s") * _NC + lax.axis_index("c")
    base = wid * _RPW
    pltpu.sync_copy(in_hbm.at[pl.ds(base, _RPW), pl.ds(_ACQ, 1), :], buf_v)
    pltpu.sync_copy(buf_v, out_hbm.at[pl.ds(base, _RPW)])


def kernel(inputs):
    return _split(inputs).reshape(_B, _D)
